# baseline (speedup 1.0000x reference)
import functools

import jax
import jax.numpy as jnp
from jax import lax
from jax.experimental import pallas as pl
from jax.experimental.pallas import tpu as pltpu
from jax.experimental.pallas import tpu_sc as plsc

VOCAB = 49408
D = 768
T = 77
B = 1024

NC, NS, L = 2, 16, 16
NW = NC * NS
NTOK = B * T
PER_W = NTOK // NW
CHUNK = 32
NCHUNK = PER_W // CHUNK
DV = D // L


def _body(tok_hbm, tab_hbm, pos_hbm, out_hbm, idx_v, buf0, buf1, pos_v,
          gsem0, gsem1, osem0, osem1):
    wid = lax.axis_index("s") * NC + lax.axis_index("c")
    base = wid * PER_W

    pltpu.sync_copy(tok_hbm.at[pl.ds(base, PER_W)], idx_v)
    pltpu.sync_copy(pos_hbm, pos_v)

    def issue(c, buf, gsem):
        pltpu.async_copy(tab_hbm.at[idx_v.at[pl.ds(c * CHUNK, CHUNK)]], buf,
                         gsem)

    def finish(c, buf, gsem, osem):
        pltpu.make_async_copy(tab_hbm.at[idx_v.at[pl.ds(0, CHUNK)]], buf,
                              gsem).wait()
        f0 = base + c * CHUNK
        t0 = lax.rem(f0, T)

        @plsc.parallel_loop(0, CHUNK, unroll=2)
        def add_row(j):
            p = lax.rem(t0 + j, T)
            for d in range(DV):
                sl = pl.ds(d * L, L)
                buf[j, sl] = buf[j, sl] + pos_v[p, sl]

        pltpu.async_copy(buf, out_hbm.at[pl.ds(f0, CHUNK)], osem)

    def wait_out(c, buf, osem):
        pltpu.make_async_copy(buf, out_hbm.at[pl.ds(base + c * CHUNK, CHUNK)],
                              osem).wait()

    issue(0, buf0, gsem0)
    issue(1, buf1, gsem1)
    finish(0, buf0, gsem0, osem0)

    def pair(g, carry):
        c1 = 2 * g + 3
        wait_out(c1 - 2, buf1, osem1)
        issue(c1, buf1, gsem1)
        finish(c1 - 1, buf0, gsem0, osem0)
        c2 = c1 + 1
        wait_out(c2 - 2, buf0, osem0)
        issue(c2, buf0, gsem0)
        finish(c2 - 1, buf1, gsem1, osem1)
        return carry

    wait_out(0, buf0, osem0)
    issue(2, buf0, gsem0)
    finish(1, buf1, gsem1, osem1)
    lax.fori_loop(0, 37, pair, 0, unroll=False)
    finish(NCHUNK - 1, buf0, gsem0, osem0)
    wait_out(NCHUNK - 2, buf1, osem1)
    wait_out(NCHUNK - 1, buf0, osem0)


@jax.jit
def _run(tokens_flat, table, pos):
    mesh = plsc.VectorSubcoreMesh(core_axis_name="c", subcore_axis_name="s")
    return pl.kernel(
        _body,
        out_type=jax.ShapeDtypeStruct((NTOK, D), jnp.float32),
        mesh=mesh,
        scratch_types=[
            pltpu.VMEM((PER_W,), jnp.int32),
            pltpu.VMEM((CHUNK, D), jnp.float32),
            pltpu.VMEM((CHUNK, D), jnp.float32),
            pltpu.VMEM((T, D), jnp.float32),
            pltpu.SemaphoreType.DMA,
            pltpu.SemaphoreType.DMA,
            pltpu.SemaphoreType.DMA,
            pltpu.SemaphoreType.DMA,
        ],
    )(tokens_flat, table, pos)


def kernel(tokens, token_embeddings, positional_embeddings):
    tokens_flat = tokens.reshape(-1).astype(jnp.int32)
    out = _run(tokens_flat, token_embeddings, positional_embeddings)
    return out.reshape(B, T, D)


# speedup vs baseline: 2.2837x; 2.2837x over previous
import jax
import jax.numpy as jnp
from jax import lax
from jax.experimental import pallas as pl
from jax.experimental.pallas import tpu as pltpu
from jax.experimental.pallas import tpu_sc as plsc

VOCAB = 49408
D = 768
T = 77
B = 1024

NC, NS, L = 2, 16, 16
NW = NC * NS
NTOK = B * T
PER_W = NTOK // NW
CHUNK = 32
NCHUNK = PER_W // CHUNK
DV = D // L


def _body(tok_hbm, tab_hbm, pos_hbm, out_hbm, idx_v, buf0, buf1, pos_v,
          gsem0, gsem1, osem0, osem1):
    wid = lax.axis_index("s") * NC + lax.axis_index("c")
    base = wid * PER_W

    pltpu.sync_copy(tok_hbm.at[pl.ds(base, PER_W)], idx_v)
    pltpu.sync_copy(pos_hbm, pos_v)

    def issue(c, buf, gsem):
        pltpu.async_copy(tab_hbm.at[idx_v.at[pl.ds(c * CHUNK, CHUNK)]], buf,
                         gsem)

    def finish(c, buf, gsem, osem):
        pltpu.make_async_copy(tab_hbm.at[idx_v.at[pl.ds(0, CHUNK)]], buf,
                              gsem).wait()
        f0 = base + c * CHUNK
        t0 = lax.shift_right_logical(f0, 10)

        @plsc.parallel_loop(0, CHUNK, unroll=2)
        def add_row(j):
            for d in range(DV):
                sl = pl.ds(d * L, L)
                buf[j, sl] = buf[j, sl] + pos_v[t0, sl]

        pltpu.async_copy(buf, out_hbm.at[pl.ds(f0, CHUNK)], osem)

    def wait_out(c, buf, osem):
        pltpu.make_async_copy(buf, out_hbm.at[pl.ds(base + c * CHUNK, CHUNK)],
                              osem).wait()

    issue(0, buf0, gsem0)
    issue(1, buf1, gsem1)
    finish(0, buf0, gsem0, osem0)

    def pair(g, carry):
        c1 = 2 * g + 3
        wait_out(c1 - 2, buf1, osem1)
        issue(c1, buf1, gsem1)
        finish(c1 - 1, buf0, gsem0, osem0)
        c2 = c1 + 1
        wait_out(c2 - 2, buf0, osem0)
        issue(c2, buf0, gsem0)
        finish(c2 - 1, buf1, gsem1, osem1)
        return carry

    wait_out(0, buf0, osem0)
    issue(2, buf0, gsem0)
    finish(1, buf1, gsem1, osem1)
    lax.fori_loop(0, 37, pair, 0, unroll=False)
    finish(NCHUNK - 1, buf0, gsem0, osem0)
    wait_out(NCHUNK - 2, buf1, osem1)
    wait_out(NCHUNK - 1, buf0, osem0)


def _run(tokens_flat_t, table, pos):
    mesh = plsc.VectorSubcoreMesh(core_axis_name="c", subcore_axis_name="s")
    return pl.kernel(
        _body,
        out_type=jax.ShapeDtypeStruct((NTOK, D), jnp.float32),
        mesh=mesh,
        scratch_types=[
            pltpu.VMEM((PER_W,), jnp.int32),
            pltpu.VMEM((CHUNK, D), jnp.float32),
            pltpu.VMEM((CHUNK, D), jnp.float32),
            pltpu.VMEM((T, D), jnp.float32),
            pltpu.SemaphoreType.DMA,
            pltpu.SemaphoreType.DMA,
            pltpu.SemaphoreType.DMA,
            pltpu.SemaphoreType.DMA,
        ],
    )(tokens_flat_t, table, pos)


def kernel(tokens, token_embeddings, positional_embeddings):
    tokens_t = jnp.transpose(tokens).reshape(-1).astype(jnp.int32)
    out = _run(tokens_t, token_embeddings, positional_embeddings)
    return jnp.transpose(out.reshape(T, B, D), (1, 0, 2))


# speedup vs baseline: 2.6043x; 1.1404x over previous
import jax
import jax.numpy as jnp
from jax import lax
from jax.experimental import pallas as pl
from jax.experimental.pallas import tpu as pltpu
from jax.experimental.pallas import tpu_sc as plsc

VOCAB = 49408
D = 768
T = 77
B = 1024

NC, NS, L = 2, 16, 16
NW = NC * NS
NTOK = B * T
PER_W = NTOK // NW
CHUNK = 32
NCHUNK = PER_W // CHUNK
DV = D // L
NBUF = 4


def _body(tok_hbm, tab_hbm, pos_hbm, out_hbm, idx_v, bufs, pbufs,
          gsems, osems, psems):
    wid = lax.axis_index("s") * NC + lax.axis_index("c")
    base = wid * PER_W

    pltpu.sync_copy(tok_hbm.at[pl.ds(base, PER_W)], idx_v)

    def issue(c, k):
        t0 = lax.shift_right_logical(base + c * CHUNK, 10)
        pltpu.async_copy(pos_hbm.at[pl.ds(t0, 1)], pbufs[k], psems[k])
        pltpu.async_copy(tab_hbm.at[idx_v.at[pl.ds(c * CHUNK, CHUNK)]],
                         bufs[k], gsems[k])

    def finish(c, k):
        buf, pbuf = bufs[k], pbufs[k]
        pltpu.make_async_copy(tab_hbm.at[idx_v.at[pl.ds(0, CHUNK)]], buf,
                              gsems[k]).wait()
        pltpu.make_async_copy(pos_hbm.at[pl.ds(0, 1)], pbuf,
                              psems[k]).wait()

        @plsc.parallel_loop(0, CHUNK, unroll=2)
        def add_row(j):
            for d in range(DV):
                sl = pl.ds(d * L, L)
                buf[j, sl] = buf[j, sl] + pbuf[0, sl]

        pltpu.async_copy(buf, out_hbm.at[pl.ds(base + c * CHUNK, CHUNK)],
                         osems[k])

    def wait_out(c, k):
        pltpu.make_async_copy(bufs[k],
                              out_hbm.at[pl.ds(base + c * CHUNK, CHUNK)],
                              osems[k]).wait()

    issue(0, 0)
    issue(1, 1)
    finish(0, 0)
    issue(2, 2)
    finish(1, 1)
    issue(3, 3)
    finish(2, 2)

    def group(g, carry):
        for k in range(NBUF):
            c = NBUF * g + k
            wait_out(c - NBUF, k)
            issue(c, k)
            finish(c - 1, (k + NBUF - 1) % NBUF)
        return carry

    lax.fori_loop(1, 19, group, 0, unroll=False)
    wait_out(72, 0)
    issue(76, 0)
    finish(75, 3)
    finish(76, 0)
    wait_out(73, 1)
    wait_out(74, 2)
    wait_out(75, 3)
    wait_out(76, 0)


def _run(tokens_flat_t, table, pos):
    mesh = plsc.VectorSubcoreMesh(core_axis_name="c", subcore_axis_name="s")

    def body(tok_hbm, tab_hbm, pos_hbm, out_hbm, idx_v,
             b0, b1, b2, b3, p0, p1, p2, p3,
             g0, g1, g2, g3, o0, o1, o2, o3, s0, s1, s2, s3):
        _body(tok_hbm, tab_hbm, pos_hbm, out_hbm, idx_v,
              (b0, b1, b2, b3), (p0, p1, p2, p3),
              (g0, g1, g2, g3), (o0, o1, o2, o3), (s0, s1, s2, s3))

    return pl.kernel(
        body,
        out_type=jax.ShapeDtypeStruct((NTOK, D), jnp.float32),
        mesh=mesh,
        scratch_types=[pltpu.VMEM((PER_W,), jnp.int32)]
        + [pltpu.VMEM((CHUNK, D), jnp.float32) for _ in range(NBUF)]
        + [pltpu.VMEM((1, D), jnp.float32) for _ in range(NBUF)]
        + [pltpu.SemaphoreType.DMA for _ in range(3 * NBUF)],
    )(tokens_flat_t, table, pos)


def kernel(tokens, token_embeddings, positional_embeddings):
    tokens_t = jnp.transpose(tokens).reshape(-1).astype(jnp.int32)
    out = _run(tokens_t, token_embeddings, positional_embeddings)
    return jnp.transpose(out.reshape(T, B, D), (1, 0, 2))
